# baseline (speedup 1.0000x reference)
import functools

import jax
import jax.numpy as jnp
from jax import lax
from jax.experimental import pallas as pl
from jax.experimental.pallas import tpu as pltpu
from jax.experimental.pallas import tpu_sc as plsc

N = 10000
E = 320000
D = 128
DE = 16

_info = plsc.get_sparse_core_info()
NC = _info.num_cores
NS = _info.num_subcores
NW = NC * NS

GW = 64
G_PER_TILE = -(-E // (GW * NW * 8)) * 8
GROUPS = G_PER_TILE * NW
E_PAD = GROUPS * GW
NBUF = 2
P0 = 320
P1 = 2 * G_PER_TILE - P0
GW2 = 32
G2_PER_TILE = E_PAD // (GW2 * NW)
GROUPS2 = G2_PER_TILE * NW
N_PAD = 10112
ROWS_PER_TILE = N_PAD // NS


def _sc_gather_scatter(nodes_p, s2d, r2d, zg):
    mesh = plsc.VectorSubcoreMesh(core_axis_name="c", subcore_axis_name="s")

    @functools.partial(
        pl.kernel,
        out_type=jax.ShapeDtypeStruct((NC, N_PAD, D), jnp.float32),
        mesh=mesh,
        scratch_types=[
            pltpu.VMEM((8, GW), jnp.int32),
            pltpu.VMEM((8, GW), jnp.int32),
        ] + [pltpu.VMEM((GW, D), jnp.float32)] * NBUF
        + [pltpu.VMEM_SHARED((N_PAD, D), jnp.float32)]
        + [pltpu.SemaphoreType.DMA] * (2 * NBUF),
    )
    def k(nodes_h, s_h, r_h, zg_h, gp_h, sidx, ridx, *rest):
        bufs = rest[:NBUF]
        acc_g = rest[NBUF]
        gsems = rest[NBUF + 1:NBUF + 1 + NBUF]
        ssems = rest[NBUF + 1 + NBUF:]
        c = lax.axis_index("c")
        s = lax.axis_index("s")
        wid = s * NC + c
        row0 = s * ROWS_PER_TILE

        pltpu.sync_copy(zg_h, acc_g.at[pl.ds(row0, ROWS_PER_TILE)])
        g_base = jnp.where(c == 0, s * P0, NS * P0 + s * P1)
        nchunks = jnp.where(c == 0, P0 // 8, P1 // 8)
        plsc.subcore_barrier()

        def outer(oj, carry):
            c_base = g_base + oj * 8
            pltpu.sync_copy(s_h.at[pl.ds(c_base, 8)], sidx)
            pltpu.sync_copy(r_h.at[pl.ds(c_base, 8)], ridx)

            gd = {}
            sd = {}
            for p in range(NBUF - 1):
                gd[p] = pltpu.async_copy(
                    nodes_h.at[sidx.at[p]], bufs[p], gsems[p])
            for j in range(8):
                b = j % NBUF
                nj = j + NBUF - 1
                if nj < 8:
                    gd[nj] = pltpu.async_copy(
                        nodes_h.at[sidx.at[nj]], bufs[nj % NBUF],
                        gsems[nj % NBUF])
                if j >= NBUF:
                    sd[j - NBUF].wait()
                gd[j].wait()
                sd[j] = pltpu.async_copy(
                    bufs[b], acc_g.at[ridx.at[j]], ssems[b], add=True)
            for j in range(8 - NBUF, 8):
                sd[j].wait()
            return carry

        lax.fori_loop(0, nchunks, outer, 0)
        plsc.subcore_barrier()

        sl = pl.ds(row0, ROWS_PER_TILE)
        pltpu.sync_copy(acc_g.at[sl], gp_h.at[c, sl])

    return k(nodes_p, s2d, r2d, zg)


def _sc_edge_deg(s2d, r2d, e3d, zg):
    mesh = plsc.VectorSubcoreMesh(core_axis_name="c", subcore_axis_name="s")

    @functools.partial(
        pl.kernel,
        out_type=jax.ShapeDtypeStruct((NC, N_PAD, D), jnp.float32),
        mesh=mesh,
        scratch_types=[
            pltpu.VMEM((16, GW2), jnp.int32),
            pltpu.VMEM((16, GW2), jnp.int32),
            pltpu.VMEM((GW2, DE), jnp.float32),
            pltpu.VMEM((GW2, DE), jnp.float32),
            pltpu.VMEM((GW2, D), jnp.float32),
            pltpu.VMEM((GW2, D), jnp.float32),
            pltpu.VMEM((GW2, D), jnp.float32),
            pltpu.VMEM_SHARED((N_PAD, D), jnp.float32),
            pltpu.SemaphoreType.DMA,
            pltpu.SemaphoreType.DMA,
            pltpu.SemaphoreType.DMA,
            pltpu.SemaphoreType.DMA,
            pltpu.SemaphoreType.DMA,
        ],
    )
    def k(s_h, r_h, e_h, zg_h, xp_h,
          sidx, ridx, ebuf_a, ebuf_b, rep_a, rep_b, smark, acc_x,
          esem_a, esem_b, rsem_a, rsem_b, msem):
        c = lax.axis_index("c")
        s = lax.axis_index("s")
        wid = s * NC + c
        row0 = s * ROWS_PER_TILE

        pltpu.sync_copy(zg_h, acc_x.at[pl.ds(row0, ROWS_PER_TILE)])

        zero16 = jnp.zeros((16,), jnp.float32)
        lane = lax.iota(jnp.int32, 16)
        marker_r = jnp.where(lane == 0, 1.0, 0.0).astype(jnp.float32)
        marker_s = jnp.where(lane == 1, 1.0, 0.0).astype(jnp.float32)

        def zrow(i, cy):
            for kk in range(8):
                rep_a[i, pl.ds(kk * 16, 16)] = zero16
                rep_b[i, pl.ds(kk * 16, 16)] = zero16
                smark[i, pl.ds(kk * 16, 16)] = zero16
            rep_a[i, pl.ds(16, 16)] = marker_r
            rep_b[i, pl.ds(16, 16)] = marker_r
            smark[i, pl.ds(16, 16)] = marker_s
            return cy

        lax.fori_loop(0, GW2, zrow, 0)

        g_base = wid * G2_PER_TILE
        plsc.subcore_barrier()

        ebufs = (ebuf_a, ebuf_b)
        reps = (rep_a, rep_b)
        esems = (esem_a, esem_b)
        rsems = (rsem_a, rsem_b)

        def outer(oj, carry):
            c_base = g_base + oj * 16
            pltpu.sync_copy(s_h.at[pl.ds(c_base, 16)], sidx)
            pltpu.sync_copy(r_h.at[pl.ds(c_base, 16)], ridx)

            ed = {}
            rd = {}
            md = {}
            ed[0] = pltpu.async_copy(e_h.at[c_base], ebufs[0], esems[0])
            for j in range(16):
                b = j % 2
                if j + 1 < 16:
                    ed[j + 1] = pltpu.async_copy(
                        e_h.at[c_base + j + 1], ebufs[(j + 1) % 2],
                        esems[(j + 1) % 2])
                if j >= 2:
                    rd[j - 2].wait()
                ed[j].wait()

                def build(bb, cy2):
                    for u in range(8):
                        reps[b][bb * 8 + u, pl.ds(0, 16)] = ebufs[b][bb * 8 + u, :]
                    return cy2

                lax.fori_loop(0, GW2 // 8, build, 0)
                rd[j] = pltpu.async_copy(
                    reps[b], acc_x.at[ridx.at[j]], rsems[b], add=True)
                md[j] = pltpu.async_copy(
                    smark, acc_x.at[sidx.at[j]], msem, add=True)
                if j >= 1:
                    md[j - 1].wait()
            rd[14].wait()
            rd[15].wait()
            md[15].wait()
            return carry

        lax.fori_loop(0, G2_PER_TILE // 16, outer, 0)
        plsc.subcore_barrier()

        sl = pl.ds(row0, ROWS_PER_TILE)
        pltpu.sync_copy(acc_x.at[sl], xp_h.at[c, sl])

    return k(s2d, r2d, e3d, zg)


_BLK = 2000


def _combine_body(gp, xp, w, we, b, out):
    g = gp[0] + gp[1]
    x = xp[0] + xp[1]
    a = x[:, 0:DE]
    cr = x[:, DE:DE + 1]
    cs = x[:, DE + 1:DE + 2]
    res = jnp.dot(g, w[...], preferred_element_type=jnp.float32)
    res = res + jnp.dot(a, we[...], preferred_element_type=jnp.float32)
    res = res + cr * b[...]
    denom = lax.rsqrt(jnp.maximum(cs, 1.0) * jnp.maximum(cr, 1.0))
    out[...] = res * denom


def _combine(gp, xp, W, We, bsum):
    grid = N // _BLK
    return pl.pallas_call(
        _combine_body,
        grid=(grid,),
        in_specs=[
            pl.BlockSpec((NC, _BLK, D), lambda i: (0, i, 0)),
            pl.BlockSpec((NC, _BLK, D), lambda i: (0, i, 0)),
            pl.BlockSpec((D, D), lambda i: (0, 0)),
            pl.BlockSpec((DE, D), lambda i: (0, 0)),
            pl.BlockSpec((1, D), lambda i: (0, 0)),
        ],
        out_specs=pl.BlockSpec((_BLK, D), lambda i: (i, 0)),
        out_shape=jax.ShapeDtypeStruct((N, D), jnp.float32),
    )(gp, xp, W, We, bsum)


def kernel(nodes, edge_attr, senders, receivers, W, bW, We, bWe):
    pad_e = E_PAD - E
    nodes_p = jnp.concatenate(
        [nodes, jnp.zeros((N_PAD - N, D), jnp.float32)], axis=0)
    s_pad = jnp.concatenate([senders, jnp.full((pad_e,), N, jnp.int32)])
    r_pad = jnp.concatenate([receivers, jnp.full((pad_e,), N, jnp.int32)])
    e_pad = jnp.concatenate([edge_attr, jnp.zeros((pad_e, DE), jnp.float32)])
    zg = jnp.zeros((ROWS_PER_TILE, D), jnp.float32)

    gp = _sc_gather_scatter(nodes_p, s_pad.reshape(GROUPS, GW),
                            r_pad.reshape(GROUPS, GW), zg)
    xp = _sc_edge_deg(s_pad.reshape(GROUPS2, GW2), r_pad.reshape(GROUPS2, GW2),
                      e_pad.reshape(GROUPS2, GW2, DE), zg)

    bsum = (bW + bWe)[None, :]
    return _combine(gp, xp, W, We, bsum)


# speedup vs baseline: 1.2279x; 1.2279x over previous
import functools

import jax
import jax.numpy as jnp
from jax import lax
from jax.experimental import pallas as pl
from jax.experimental.pallas import tpu as pltpu
from jax.experimental.pallas import tpu_sc as plsc

N = 10000
E = 320000
D = 128
DE = 16

_info = plsc.get_sparse_core_info()
NC = _info.num_cores
NS = _info.num_subcores
NW = NC * NS

GW = 64
G_PER_TILE = -(-E // (GW * NW * 8)) * 8
GROUPS = G_PER_TILE * NW
E_PAD = GROUPS * GW
NBUF = 2
P0 = 288
P1 = 2 * G_PER_TILE - P0
GW2 = 32
G2_PER_TILE = E_PAD // (GW2 * NW)
GROUPS2 = G2_PER_TILE * NW
N_PAD = 10112
ROWS_PER_TILE = N_PAD // NS


def _sc_gather_scatter(nodes_p, s2d, r2d, zg):
    mesh = plsc.VectorSubcoreMesh(core_axis_name="c", subcore_axis_name="s")

    @functools.partial(
        pl.kernel,
        out_type=jax.ShapeDtypeStruct((NC, N_PAD, D), jnp.float32),
        mesh=mesh,
        scratch_types=[
            pltpu.VMEM((8, GW), jnp.int32),
            pltpu.VMEM((8, GW), jnp.int32),
        ] + [pltpu.VMEM((GW, D), jnp.float32)] * NBUF
        + [pltpu.VMEM_SHARED((N_PAD, D), jnp.float32)]
        + [pltpu.SemaphoreType.DMA] * (2 * NBUF),
    )
    def k(nodes_h, s_h, r_h, zg_h, gp_h, sidx, ridx, *rest):
        bufs = rest[:NBUF]
        acc_g = rest[NBUF]
        gsems = rest[NBUF + 1:NBUF + 1 + NBUF]
        ssems = rest[NBUF + 1 + NBUF:]
        c = lax.axis_index("c")
        s = lax.axis_index("s")
        wid = s * NC + c
        row0 = s * ROWS_PER_TILE

        pltpu.sync_copy(zg_h, acc_g.at[pl.ds(row0, ROWS_PER_TILE)])
        g_base = jnp.where(c == 0, s * P0, NS * P0 + s * P1)
        nchunks = jnp.where(c == 0, P0 // 8, P1 // 8)
        plsc.subcore_barrier()

        def outer(oj, carry):
            c_base = g_base + oj * 8
            pltpu.sync_copy(s_h.at[pl.ds(c_base, 8)], sidx)
            pltpu.sync_copy(r_h.at[pl.ds(c_base, 8)], ridx)

            gd = {}
            sd = {}
            for p in range(NBUF - 1):
                gd[p] = pltpu.async_copy(
                    nodes_h.at[sidx.at[p]], bufs[p], gsems[p])
            for j in range(8):
                b = j % NBUF
                nj = j + NBUF - 1
                if nj < 8:
                    gd[nj] = pltpu.async_copy(
                        nodes_h.at[sidx.at[nj]], bufs[nj % NBUF],
                        gsems[nj % NBUF])
                if j >= NBUF:
                    sd[j - NBUF].wait()
                gd[j].wait()
                sd[j] = pltpu.async_copy(
                    bufs[b], acc_g.at[ridx.at[j]], ssems[b], add=True)
            for j in range(8 - NBUF, 8):
                sd[j].wait()
            return carry

        lax.fori_loop(0, nchunks, outer, 0)
        plsc.subcore_barrier()

        sl = pl.ds(row0, ROWS_PER_TILE)
        pltpu.sync_copy(acc_g.at[sl], gp_h.at[c, sl])

    return k(nodes_p, s2d, r2d, zg)


def _sc_edge_deg(s2d, r2d, e3d, zg):
    mesh = plsc.VectorSubcoreMesh(core_axis_name="c", subcore_axis_name="s")

    @functools.partial(
        pl.kernel,
        out_type=jax.ShapeDtypeStruct((NC, N_PAD, D), jnp.float32),
        mesh=mesh,
        scratch_types=[
            pltpu.VMEM((16, GW2), jnp.int32),
            pltpu.VMEM((16, GW2), jnp.int32),
            pltpu.VMEM((GW2, DE), jnp.float32),
            pltpu.VMEM((GW2, DE), jnp.float32),
            pltpu.VMEM((GW2, D), jnp.float32),
            pltpu.VMEM((GW2, D), jnp.float32),
            pltpu.VMEM((GW2, D), jnp.float32),
            pltpu.VMEM_SHARED((N_PAD, D), jnp.float32),
            pltpu.SemaphoreType.DMA,
            pltpu.SemaphoreType.DMA,
            pltpu.SemaphoreType.DMA,
            pltpu.SemaphoreType.DMA,
            pltpu.SemaphoreType.DMA,
        ],
    )
    def k(s_h, r_h, e_h, zg_h, xp_h,
          sidx, ridx, ebuf_a, ebuf_b, rep_a, rep_b, smark, acc_x,
          esem_a, esem_b, rsem_a, rsem_b, msem):
        c = lax.axis_index("c")
        s = lax.axis_index("s")
        wid = s * NC + c
        row0 = s * ROWS_PER_TILE

        pltpu.sync_copy(zg_h, acc_x.at[pl.ds(row0, ROWS_PER_TILE)])

        zero16 = jnp.zeros((16,), jnp.float32)
        lane = lax.iota(jnp.int32, 16)
        marker_r = jnp.where(lane == 0, 1.0, 0.0).astype(jnp.float32)
        marker_s = jnp.where(lane == 1, 1.0, 0.0).astype(jnp.float32)

        def zrow(i, cy):
            for kk in range(8):
                rep_a[i, pl.ds(kk * 16, 16)] = zero16
                rep_b[i, pl.ds(kk * 16, 16)] = zero16
                smark[i, pl.ds(kk * 16, 16)] = zero16
            rep_a[i, pl.ds(16, 16)] = marker_r
            rep_b[i, pl.ds(16, 16)] = marker_r
            smark[i, pl.ds(16, 16)] = marker_s
            return cy

        lax.fori_loop(0, GW2, zrow, 0)

        g_base = wid * G2_PER_TILE
        plsc.subcore_barrier()

        ebufs = (ebuf_a, ebuf_b)
        reps = (rep_a, rep_b)
        esems = (esem_a, esem_b)
        rsems = (rsem_a, rsem_b)

        def outer(oj, carry):
            c_base = g_base + oj * 16
            pltpu.sync_copy(s_h.at[pl.ds(c_base, 16)], sidx)
            pltpu.sync_copy(r_h.at[pl.ds(c_base, 16)], ridx)

            ed = {}
            rd = {}
            md = {}
            ed[0] = pltpu.async_copy(e_h.at[c_base], ebufs[0], esems[0])
            for j in range(16):
                b = j % 2
                if j + 1 < 16:
                    ed[j + 1] = pltpu.async_copy(
                        e_h.at[c_base + j + 1], ebufs[(j + 1) % 2],
                        esems[(j + 1) % 2])
                if j >= 2:
                    rd[j - 2].wait()
                ed[j].wait()

                def build(bb, cy2):
                    for u in range(8):
                        reps[b][bb * 8 + u, pl.ds(0, 16)] = ebufs[b][bb * 8 + u, :]
                    return cy2

                lax.fori_loop(0, GW2 // 8, build, 0)
                rd[j] = pltpu.async_copy(
                    reps[b], acc_x.at[ridx.at[j]], rsems[b], add=True)
                md[j] = pltpu.async_copy(
                    smark, acc_x.at[sidx.at[j]], msem, add=True)
                if j >= 1:
                    md[j - 1].wait()
            rd[14].wait()
            rd[15].wait()
            md[15].wait()
            return carry

        lax.fori_loop(0, G2_PER_TILE // 16, outer, 0)
        plsc.subcore_barrier()

        sl = pl.ds(row0, ROWS_PER_TILE)
        pltpu.sync_copy(acc_x.at[sl], xp_h.at[c, sl])

    return k(s2d, r2d, e3d, zg)


_BLK = 2000


def _combine_body(gp, xp, w, we, b, out):
    g = gp[0] + gp[1]
    x = xp[0] + xp[1]
    a = x[:, 0:DE]
    cr = x[:, DE:DE + 1]
    cs = x[:, DE + 1:DE + 2]
    res = jnp.dot(g, w[...], preferred_element_type=jnp.float32)
    res = res + jnp.dot(a, we[...], preferred_element_type=jnp.float32)
    res = res + cr * b[...]
    denom = lax.rsqrt(jnp.maximum(cs, 1.0) * jnp.maximum(cr, 1.0))
    out[...] = res * denom


def _combine(gp, xp, W, We, bsum):
    grid = N // _BLK
    return pl.pallas_call(
        _combine_body,
        grid=(grid,),
        in_specs=[
            pl.BlockSpec((NC, _BLK, D), lambda i: (0, i, 0)),
            pl.BlockSpec((NC, _BLK, D), lambda i: (0, i, 0)),
            pl.BlockSpec((D, D), lambda i: (0, 0)),
            pl.BlockSpec((DE, D), lambda i: (0, 0)),
            pl.BlockSpec((1, D), lambda i: (0, 0)),
        ],
        out_specs=pl.BlockSpec((_BLK, D), lambda i: (i, 0)),
        out_shape=jax.ShapeDtypeStruct((N, D), jnp.float32),
    )(gp, xp, W, We, bsum)


def kernel(nodes, edge_attr, senders, receivers, W, bW, We, bWe):
    pad_e = E_PAD - E
    nodes_p = jnp.concatenate(
        [nodes, jnp.zeros((N_PAD - N, D), jnp.float32)], axis=0)
    s_pad = jnp.concatenate([senders, jnp.full((pad_e,), N, jnp.int32)])
    r_pad = jnp.concatenate([receivers, jnp.full((pad_e,), N, jnp.int32)])
    e_pad = jnp.concatenate([edge_attr, jnp.zeros((pad_e, DE), jnp.float32)])
    zg = jnp.zeros((ROWS_PER_TILE, D), jnp.float32)

    gp = _sc_gather_scatter(nodes_p, s_pad.reshape(GROUPS, GW),
                            r_pad.reshape(GROUPS, GW), zg)
    xp = _sc_edge_deg(s_pad.reshape(GROUPS2, GW2), r_pad.reshape(GROUPS2, GW2),
                      e_pad.reshape(GROUPS2, GW2, DE), zg)

    bsum = (bW + bWe)[None, :]
    return _combine(gp, xp, W, We, bsum)
